# baseline (speedup 1.0000x reference)
import jax
import jax.numpy as jnp
from jax import lax
from jax.experimental import pallas as pl
from jax.experimental.pallas import tpu as pltpu
from jax.experimental.pallas import tpu_sc as plsc

_N = 10000
_E = 320000
_D = 128
_DE = 16
_H = 128

_NC = 2
_NS = 16
_NW = _NC * _NS
_EW = _E // _NW
_CH = 80
_NCH = _EW // _CH
_RPT = 624
_RREM = _N - _NS * _RPT
_SS = 25
_NSS = _NCH // _SS

_mesh = plsc.VectorSubcoreMesh(core_axis_name="c", subcore_axis_name="s")


def _deg_body(dst_hbm, out_hbm, idx_v, acc_v):
    w = lax.axis_index("s") * _NC + lax.axis_index("c")
    zeros = jnp.zeros((16,), jnp.float32)

    def zero(i, c):
        acc_v[pl.ds(i * 16, 16)] = zeros
        return c

    lax.fori_loop(0, _N // 16, zero, 0)
    pltpu.sync_copy(dst_hbm.at[w], idx_v)
    ones = jnp.ones((16,), jnp.float32)

    def body(i, c):
        idx = idx_v[pl.ds(i * 16, 16)]
        plsc.addupdate_scatter(acc_v, [idx], ones)
        return c

    lax.fori_loop(0, _EW // 16, body, 0)
    pltpu.sync_copy(acc_v, out_hbm.at[w])


_deg_call = pl.kernel(
    _deg_body,
    out_type=jax.ShapeDtypeStruct((_NW, _N), jnp.float32),
    mesh=_mesh,
    compiler_params=pltpu.CompilerParams(needs_layout_passes=False),
    scratch_types=[
        pltpu.VMEM((_EW,), jnp.int32),
        pltpu.VMEM((_N,), jnp.float32),
    ],
)


def _seg_body(y_hbm, z_hbm, sidx_hbm, didx_hbm, out_hbm,
              sidx_v, didx_v, buf0, buf1, acc_sh, sem0, sem1):
    c = lax.axis_index("c")
    s = lax.axis_index("s")
    w = s * _NC + c
    r0 = s * _RPT

    @pl.when(c == 0)
    def _():
        pltpu.sync_copy(y_hbm.at[pl.ds(r0, _RPT)], acc_sh.at[pl.ds(r0, _RPT)])

    @pl.when(c != 0)
    def _():
        pltpu.sync_copy(z_hbm.at[pl.ds(r0, _RPT)], acc_sh.at[pl.ds(r0, _RPT)])

    rtail = _NS * _RPT

    @pl.when((c == 0) & (s == 0))
    def _():
        pltpu.sync_copy(y_hbm.at[pl.ds(rtail, _RREM)],
                        acc_sh.at[pl.ds(rtail, _RREM)])

    @pl.when((c != 0) & (s == 0))
    def _():
        pltpu.sync_copy(z_hbm.at[pl.ds(rtail, _RREM)],
                        acc_sh.at[pl.ds(rtail, _RREM)])

    plsc.subcore_barrier()

    def gath(j, buf, sem):
        pltpu.async_copy(y_hbm.at[sidx_v.at[j]], buf, sem)

    def wait(j, buf, sem):
        pltpu.make_async_copy(y_hbm.at[sidx_v.at[j]], buf, sem).wait()

    def scat(j, buf):
        pltpu.sync_copy(buf, acc_sh.at[didx_v.at[j]], add=True)

    def super_body(ss, carry):
        pltpu.sync_copy(sidx_hbm.at[w, ss], sidx_v)
        pltpu.sync_copy(didx_hbm.at[w, ss], didx_v)
        gath(0, buf0, sem0)

        def body(k, c2):
            j = 2 * k
            gath(j + 1, buf1, sem1)
            wait(j, buf0, sem0)
            scat(j, buf0)
            gath(j + 2, buf0, sem0)
            wait(j + 1, buf1, sem1)
            scat(j + 1, buf1)
            return c2

        lax.fori_loop(0, (_SS - 1) // 2, body, 0)
        wait(_SS - 1, buf0, sem0)
        scat(_SS - 1, buf0)
        return carry

    lax.fori_loop(0, _NSS, super_body, 0)
    plsc.subcore_barrier()
    pltpu.sync_copy(acc_sh.at[pl.ds(r0, _RPT)], out_hbm.at[c, pl.ds(r0, _RPT)])

    @pl.when(s == 0)
    def _():
        pltpu.sync_copy(acc_sh.at[pl.ds(rtail, _RREM)],
                        out_hbm.at[c, pl.ds(rtail, _RREM)])


_seg_call = pl.kernel(
    _seg_body,
    out_type=jax.ShapeDtypeStruct((_NC, _N, _H), jnp.float32),
    mesh=_mesh,
    scratch_types=[
        pltpu.VMEM((_SS, _CH), jnp.int32),
        pltpu.VMEM((_SS, _CH), jnp.int32),
        pltpu.VMEM((_CH, _H), jnp.float32),
        pltpu.VMEM((_CH, _H), jnp.float32),
        pltpu.VMEM_SHARED((_N, _H), jnp.float32),
        pltpu.SemaphoreType.DMA,
        pltpu.SemaphoreType.DMA,
    ],
)


def _edge_body(a_hbm, b_hbm, sidx_hbm, didx_hbm, out_hbm,
               sidx_v, didx_v, buf0, buf1, sa0, sa1, sb0, sb1):
    w = lax.axis_index("s") * _NC + lax.axis_index("c")
    pltpu.sync_copy(sidx_hbm.at[w], sidx_v)
    pltpu.sync_copy(didx_hbm.at[w], didx_v)
    base = w * _NCH

    def ga(j, buf, sem):
        pltpu.async_copy(a_hbm.at[sidx_v.at[j]], buf, sem)

    def wa(j, buf, sem):
        pltpu.make_async_copy(a_hbm.at[sidx_v.at[j]], buf, sem).wait()

    def gb(j, buf, sem):
        pltpu.async_copy(b_hbm.at[didx_v.at[j]], buf, sem, add=True)

    def wb(j, buf, sem):
        pltpu.make_async_copy(b_hbm.at[didx_v.at[j]], buf, sem).wait()

    def wr(j, buf):
        pltpu.sync_copy(buf, out_hbm.at[base + j])

    ga(0, buf0, sa0)
    wa(0, buf0, sa0)
    gb(0, buf0, sb0)
    ga(1, buf1, sa1)

    def body(k, carry):
        j = 2 * k + 1
        wb(j - 1, buf0, sb0)
        wr(j - 1, buf0)
        ga(j + 1, buf0, sa0)
        wa(j, buf1, sa1)
        gb(j, buf1, sb1)
        wb(j, buf1, sb1)
        wr(j, buf1)
        ga(j + 2, buf1, sa1)
        wa(j + 1, buf0, sa0)
        gb(j + 1, buf0, sb0)
        return carry

    lax.fori_loop(0, (_NCH - 3) // 2, body, 0)
    j = _NCH - 2
    wb(j - 1, buf0, sb0)
    wr(j - 1, buf0)
    ga(j + 1, buf0, sa0)
    wa(j, buf1, sa1)
    gb(j, buf1, sb1)
    j = _NCH - 1
    wb(j - 1, buf1, sb1)
    wr(j - 1, buf1)
    wa(j, buf0, sa0)
    gb(j, buf0, sb0)
    wb(j, buf0, sb0)
    wr(j, buf0)


_edge_call = pl.kernel(
    _edge_body,
    out_type=jax.ShapeDtypeStruct((_NW * _NCH, _CH, _H), jnp.float32),
    mesh=_mesh,
    scratch_types=[
        pltpu.VMEM((_NCH, _CH), jnp.int32),
        pltpu.VMEM((_NCH, _CH), jnp.int32),
        pltpu.VMEM((_CH, _H), jnp.float32),
        pltpu.VMEM((_CH, _H), jnp.float32),
        pltpu.SemaphoreType.DMA,
        pltpu.SemaphoreType.DMA,
        pltpu.SemaphoreType.DMA,
        pltpu.SemaphoreType.DMA,
    ],
)


_RB = 2000
_RBE = 8192


def _tc1_body(x_ref, w_ref, degt_ref, y_ref, dinv_ref):
    deg = jnp.sum(degt_ref[...], axis=1) + 1.0
    dinv = lax.rsqrt(deg)[:, None]
    dinv_ref[...] = dinv
    xw = jnp.dot(x_ref[...], w_ref[...], preferred_element_type=jnp.float32)
    y_ref[...] = xw * dinv


_tc1 = pl.pallas_call(
    _tc1_body,
    grid=(_N // _RB,),
    in_specs=[
        pl.BlockSpec((_RB, _D), lambda i: (i, 0)),
        pl.BlockSpec((_D, _H), lambda i: (0, 0)),
        pl.BlockSpec((_RB, _NW), lambda i: (i, 0)),
    ],
    out_specs=[
        pl.BlockSpec((_RB, _H), lambda i: (i, 0)),
        pl.BlockSpec((_RB, 1), lambda i: (i, 0)),
    ],
    out_shape=[
        jax.ShapeDtypeStruct((_N, _H), jnp.float32),
        jax.ShapeDtypeStruct((_N, 1), jnp.float32),
    ],
)


def _tc2_body(p_ref, dinv_ref, b1_ref, w2_ref, y2_ref):
    dinv = dinv_ref[...]
    h1 = jnp.maximum((p_ref[0] + p_ref[1]) * dinv + b1_ref[...], 0.0)
    y2_ref[...] = jnp.dot(h1, w2_ref[...],
                          preferred_element_type=jnp.float32) * dinv


_tc2 = pl.pallas_call(
    _tc2_body,
    grid=(_N // _RB,),
    in_specs=[
        pl.BlockSpec((_NC, _RB, _H), lambda i: (0, i, 0)),
        pl.BlockSpec((_RB, 1), lambda i: (i, 0)),
        pl.BlockSpec((_H,), lambda i: (0,)),
        pl.BlockSpec((_H, _H), lambda i: (0, 0)),
    ],
    out_specs=pl.BlockSpec((_RB, _H), lambda i: (i, 0)),
    out_shape=jax.ShapeDtypeStruct((_N, _H), jnp.float32),
)


def _tc3_body(q_ref, dinv_ref, b2_ref, wa_ref, wb_ref, bm1_ref, a_ref, b_ref):
    dinv = dinv_ref[...]
    h = (q_ref[0] + q_ref[1]) * dinv + b2_ref[...]
    a_ref[...] = jnp.dot(h, wa_ref[...], preferred_element_type=jnp.float32)
    b_ref[...] = jnp.dot(h, wb_ref[...],
                         preferred_element_type=jnp.float32) + bm1_ref[...]


_tc3 = pl.pallas_call(
    _tc3_body,
    grid=(_N // _RB,),
    in_specs=[
        pl.BlockSpec((_NC, _RB, _H), lambda i: (0, i, 0)),
        pl.BlockSpec((_RB, 1), lambda i: (i, 0)),
        pl.BlockSpec((_H,), lambda i: (0,)),
        pl.BlockSpec((_H, _H), lambda i: (0, 0)),
        pl.BlockSpec((_H, _H), lambda i: (0, 0)),
        pl.BlockSpec((_H,), lambda i: (0,)),
    ],
    out_specs=[
        pl.BlockSpec((_RB, _H), lambda i: (i, 0)),
        pl.BlockSpec((_RB, _H), lambda i: (i, 0)),
    ],
    out_shape=[
        jax.ShapeDtypeStruct((_N, _H), jnp.float32),
        jax.ShapeDtypeStruct((_N, _H), jnp.float32),
    ],
)


_B0 = 1024


def _tc4_body(t3_ref, ea8_ref, wc_ref, w2_ref, bm2_ref, o_ref):
    ea = ea8_ref[...]
    wc = wc_ref[...]
    w2 = w2_ref[...]
    for j in range(8):
        cj = jnp.dot(ea[:, 16 * j:16 * j + 16], wc,
                     preferred_element_type=jnp.float32)
        zj = jnp.maximum(t3_ref[:, j, :] + cj, 0.0)
        sj = jnp.sum(zj * w2, axis=1) + bm2_ref[...]
        o_ref[j, :] = 1.0 / (1.0 + jnp.exp(-sj))


_tc4 = pl.pallas_call(
    _tc4_body,
    grid=(pl.cdiv(_E // 8, _B0),),
    in_specs=[
        pl.BlockSpec((_B0, 8, _H), lambda i: (i, 0, 0)),
        pl.BlockSpec((_B0, _H), lambda i: (i, 0)),
        pl.BlockSpec((_DE, _H), lambda i: (0, 0)),
        pl.BlockSpec((1, _H), lambda i: (0, 0)),
        pl.BlockSpec((1,), lambda i: (0,)),
    ],
    out_specs=pl.BlockSpec((8, _B0), lambda i: (0, i)),
    out_shape=jax.ShapeDtypeStruct((8, _E // 8), jnp.float32),
)


def kernel(x, edge_index, edge_attr, W1, b1, W2, b2, Wm1, bm1, Wm2, bm2):
    src = edge_index[0]
    dst = edge_index[1]
    src3 = src.reshape(_NW, _NCH, _CH)
    dst3 = dst.reshape(_NW, _NCH, _CH)
    src4 = src.reshape(_NW, _NSS, _SS, _CH)
    dst4 = dst.reshape(_NW, _NSS, _SS, _CH)
    dst2 = dst.reshape(_NW, _EW)
    zeros_nh = jnp.zeros((_N, _H), jnp.float32)

    degp = _deg_call(dst2)
    y1, dinv = _tc1(x, W1, degp.T)
    p1 = _seg_call(y1, zeros_nh, src4, dst4)
    y2 = _tc2(p1, dinv, b1, W2)
    p2 = _seg_call(y2, zeros_nh, src4, dst4)
    a_t, b_t = _tc3(p2, dinv, b2, Wm1[:_H], Wm1[_H:2 * _H], bm1)
    t3 = _edge_call(a_t, b_t, src3, dst3).reshape(_E // 8, 8, _H)
    ea8 = edge_attr.reshape(_E // 8, 8 * _DE)
    out8 = _tc4(t3, ea8, Wm1[2 * _H:], Wm2.reshape(1, _H), bm2)
    return out8.T.reshape(_E)


# speedup vs baseline: 1.4046x; 1.4046x over previous
import jax
import jax.numpy as jnp
from jax import lax
from jax.experimental import pallas as pl
from jax.experimental.pallas import tpu as pltpu
from jax.experimental.pallas import tpu_sc as plsc

_N = 10000
_E = 320000
_D = 128
_DE = 16
_H = 128

_NC = 2
_NS = 16
_NW = _NC * _NS
_EW = _E // _NW
_CH = 80
_NCH = _EW // _CH
_RPT = 624
_RREM = _N - _NS * _RPT
_SS = 25
_NSS = _NCH // _SS

_mesh = plsc.VectorSubcoreMesh(core_axis_name="c", subcore_axis_name="s")


def _deg_body(dst_hbm, out_hbm, idx_v, acc_v):
    w = lax.axis_index("s") * _NC + lax.axis_index("c")
    zeros = jnp.zeros((16,), jnp.float32)

    def zero(i, c):
        acc_v[pl.ds(i * 16, 16)] = zeros
        return c

    lax.fori_loop(0, _N // 16, zero, 0)
    pltpu.sync_copy(dst_hbm.at[w], idx_v)
    ones = jnp.ones((16,), jnp.float32)

    def body(i, c):
        idx = idx_v[pl.ds(i * 16, 16)]
        plsc.addupdate_scatter(acc_v, [idx], ones)
        return c

    lax.fori_loop(0, _EW // 16, body, 0)
    pltpu.sync_copy(acc_v, out_hbm.at[w])


_deg_call = pl.kernel(
    _deg_body,
    out_type=jax.ShapeDtypeStruct((_NW, _N), jnp.float32),
    mesh=_mesh,
    compiler_params=pltpu.CompilerParams(needs_layout_passes=False),
    scratch_types=[
        pltpu.VMEM((_EW,), jnp.int32),
        pltpu.VMEM((_N,), jnp.float32),
    ],
)


def _seg_body(y_hbm, z_hbm, sidx_hbm, didx_hbm, out_hbm,
              sidx_v, didx_v, buf0, buf1, acc_sh, sem0, sem1):
    c = lax.axis_index("c")
    s = lax.axis_index("s")
    w = s * _NC + c
    r0 = s * _RPT

    @pl.when(c == 0)
    def _():
        pltpu.sync_copy(y_hbm.at[pl.ds(r0, _RPT)], acc_sh.at[pl.ds(r0, _RPT)])

    @pl.when(c != 0)
    def _():
        pltpu.sync_copy(z_hbm.at[pl.ds(r0, _RPT)], acc_sh.at[pl.ds(r0, _RPT)])

    rtail = _NS * _RPT

    @pl.when((c == 0) & (s == 0))
    def _():
        pltpu.sync_copy(y_hbm.at[pl.ds(rtail, _RREM)],
                        acc_sh.at[pl.ds(rtail, _RREM)])

    @pl.when((c != 0) & (s == 0))
    def _():
        pltpu.sync_copy(z_hbm.at[pl.ds(rtail, _RREM)],
                        acc_sh.at[pl.ds(rtail, _RREM)])

    plsc.subcore_barrier()

    def gath(j, buf, sem):
        pltpu.async_copy(y_hbm.at[sidx_v.at[j]], buf, sem)

    def wait(j, buf, sem):
        pltpu.make_async_copy(y_hbm.at[sidx_v.at[j]], buf, sem).wait()

    def scat(j, buf):
        pltpu.sync_copy(buf, acc_sh.at[didx_v.at[j]], add=True)

    def super_body(ss, carry):
        pltpu.sync_copy(sidx_hbm.at[w, ss], sidx_v)
        pltpu.sync_copy(didx_hbm.at[w, ss], didx_v)
        gath(0, buf0, sem0)

        def body(k, c2):
            j = 2 * k
            gath(j + 1, buf1, sem1)
            wait(j, buf0, sem0)
            scat(j, buf0)
            gath(j + 2, buf0, sem0)
            wait(j + 1, buf1, sem1)
            scat(j + 1, buf1)
            return c2

        lax.fori_loop(0, (_SS - 1) // 2, body, 0)
        wait(_SS - 1, buf0, sem0)
        scat(_SS - 1, buf0)
        return carry

    lax.fori_loop(0, _NSS, super_body, 0)
    plsc.subcore_barrier()
    pltpu.sync_copy(acc_sh.at[pl.ds(r0, _RPT)], out_hbm.at[c, pl.ds(r0, _RPT)])

    @pl.when(s == 0)
    def _():
        pltpu.sync_copy(acc_sh.at[pl.ds(rtail, _RREM)],
                        out_hbm.at[c, pl.ds(rtail, _RREM)])


_seg_call = pl.kernel(
    _seg_body,
    out_type=jax.ShapeDtypeStruct((_NC, _N, _H), jnp.float32),
    mesh=_mesh,
    scratch_types=[
        pltpu.VMEM((_SS, _CH), jnp.int32),
        pltpu.VMEM((_SS, _CH), jnp.int32),
        pltpu.VMEM((_CH, _H), jnp.float32),
        pltpu.VMEM((_CH, _H), jnp.float32),
        pltpu.VMEM_SHARED((_N, _H), jnp.float32),
        pltpu.SemaphoreType.DMA,
        pltpu.SemaphoreType.DMA,
    ],
)


def _edge_body(a_hbm, b_hbm, sidx_hbm, didx_hbm, out_hbm,
               sidx_v, didx_v, b0, b1, b2, b3,
               sa0, sa1, sa2, sa3, sb0, sb1, sb2, sb3):
    w = lax.axis_index("s") * _NC + lax.axis_index("c")
    pltpu.sync_copy(sidx_hbm.at[w], sidx_v)
    pltpu.sync_copy(didx_hbm.at[w], didx_v)
    base = w * _NCH
    bufs = (b0, b1, b2, b3)
    sas = (sa0, sa1, sa2, sa3)
    sbs = (sb0, sb1, sb2, sb3)

    def ga(j, i, sem=None):
        pltpu.async_copy(a_hbm.at[sidx_v.at[j]], bufs[i], sas[i])

    def wa(j, i):
        pltpu.make_async_copy(a_hbm.at[sidx_v.at[j]], bufs[i], sas[i]).wait()

    def gb(j, i):
        pltpu.async_copy(b_hbm.at[didx_v.at[j]], bufs[i], sbs[i], add=True)

    def wb(j, i):
        pltpu.make_async_copy(b_hbm.at[didx_v.at[j]], bufs[i], sbs[i]).wait()

    def wr(j, i):
        pltpu.sync_copy(bufs[i], out_hbm.at[base + j])

    ga(0, 0)
    ga(1, 1)
    wa(0, 0)
    gb(0, 0)
    ga(2, 2)
    wa(1, 1)
    gb(1, 1)
    ga(3, 3)

    def body(k, carry):
        jb = 4 * k + 2
        for o in range(4):
            j = jb + o
            y = (o + 2) % 4
            wb(j - 2, o)
            wr(j - 2, o)
            ga(j + 2, o)
            wa(j, y)
            gb(j, y)
        return carry

    lax.fori_loop(0, (_NCH - 5) // 4, body, 0)
    j = _NCH - 3
    wb(j - 2, 0)
    wr(j - 2, 0)
    ga(j + 2, 0)
    wa(j, 2)
    gb(j, 2)
    j = _NCH - 2
    wb(j - 2, 1)
    wr(j - 2, 1)
    wa(j, 3)
    gb(j, 3)
    j = _NCH - 1
    wb(j - 2, 2)
    wr(j - 2, 2)
    wa(j, 0)
    gb(j, 0)
    wb(_NCH - 2, 3)
    wr(_NCH - 2, 3)
    wb(_NCH - 1, 0)
    wr(_NCH - 1, 0)


_edge_call = pl.kernel(
    _edge_body,
    out_type=jax.ShapeDtypeStruct((_NW * _NCH, _CH, _H), jnp.float32),
    mesh=_mesh,
    scratch_types=[
        pltpu.VMEM((_NCH, _CH), jnp.int32),
        pltpu.VMEM((_NCH, _CH), jnp.int32),
        pltpu.VMEM((_CH, _H), jnp.float32),
        pltpu.VMEM((_CH, _H), jnp.float32),
        pltpu.VMEM((_CH, _H), jnp.float32),
        pltpu.VMEM((_CH, _H), jnp.float32),
        pltpu.SemaphoreType.DMA,
        pltpu.SemaphoreType.DMA,
        pltpu.SemaphoreType.DMA,
        pltpu.SemaphoreType.DMA,
        pltpu.SemaphoreType.DMA,
        pltpu.SemaphoreType.DMA,
        pltpu.SemaphoreType.DMA,
        pltpu.SemaphoreType.DMA,
    ],
)


_RB = 2000
_RBE = 8192


def _tc1_body(x_ref, w_ref, degt_ref, y_ref, dinv_ref):
    deg = jnp.sum(degt_ref[...], axis=1) + 1.0
    dinv = lax.rsqrt(deg)[:, None]
    dinv_ref[...] = dinv
    xw = jnp.dot(x_ref[...], w_ref[...], preferred_element_type=jnp.float32)
    y_ref[...] = xw * dinv


_tc1 = pl.pallas_call(
    _tc1_body,
    grid=(_N // _RB,),
    in_specs=[
        pl.BlockSpec((_RB, _D), lambda i: (i, 0)),
        pl.BlockSpec((_D, _H), lambda i: (0, 0)),
        pl.BlockSpec((_RB, _NW), lambda i: (i, 0)),
    ],
    out_specs=[
        pl.BlockSpec((_RB, _H), lambda i: (i, 0)),
        pl.BlockSpec((_RB, 1), lambda i: (i, 0)),
    ],
    out_shape=[
        jax.ShapeDtypeStruct((_N, _H), jnp.float32),
        jax.ShapeDtypeStruct((_N, 1), jnp.float32),
    ],
)


def _tc2_body(p_ref, dinv_ref, b1_ref, w2_ref, y2_ref):
    dinv = dinv_ref[...]
    h1 = jnp.maximum((p_ref[0] + p_ref[1]) * dinv + b1_ref[...], 0.0)
    y2_ref[...] = jnp.dot(h1, w2_ref[...],
                          preferred_element_type=jnp.float32) * dinv


_tc2 = pl.pallas_call(
    _tc2_body,
    grid=(_N // _RB,),
    in_specs=[
        pl.BlockSpec((_NC, _RB, _H), lambda i: (0, i, 0)),
        pl.BlockSpec((_RB, 1), lambda i: (i, 0)),
        pl.BlockSpec((_H,), lambda i: (0,)),
        pl.BlockSpec((_H, _H), lambda i: (0, 0)),
    ],
    out_specs=pl.BlockSpec((_RB, _H), lambda i: (i, 0)),
    out_shape=jax.ShapeDtypeStruct((_N, _H), jnp.float32),
)


def _tc3_body(q_ref, dinv_ref, b2_ref, wa_ref, wb_ref, bm1_ref, a_ref, b_ref):
    dinv = dinv_ref[...]
    h = (q_ref[0] + q_ref[1]) * dinv + b2_ref[...]
    a_ref[...] = jnp.dot(h, wa_ref[...], preferred_element_type=jnp.float32)
    b_ref[...] = jnp.dot(h, wb_ref[...],
                         preferred_element_type=jnp.float32) + bm1_ref[...]


_tc3 = pl.pallas_call(
    _tc3_body,
    grid=(_N // _RB,),
    in_specs=[
        pl.BlockSpec((_NC, _RB, _H), lambda i: (0, i, 0)),
        pl.BlockSpec((_RB, 1), lambda i: (i, 0)),
        pl.BlockSpec((_H,), lambda i: (0,)),
        pl.BlockSpec((_H, _H), lambda i: (0, 0)),
        pl.BlockSpec((_H, _H), lambda i: (0, 0)),
        pl.BlockSpec((_H,), lambda i: (0,)),
    ],
    out_specs=[
        pl.BlockSpec((_RB, _H), lambda i: (i, 0)),
        pl.BlockSpec((_RB, _H), lambda i: (i, 0)),
    ],
    out_shape=[
        jax.ShapeDtypeStruct((_N, _H), jnp.float32),
        jax.ShapeDtypeStruct((_N, _H), jnp.float32),
    ],
)


def _tc4_body(t_ref, ea_ref, wc_ref, w2_ref, bm2_ref, o_ref):
    c = jnp.dot(ea_ref[...], wc_ref[...], preferred_element_type=jnp.float32)
    z = jnp.maximum(t_ref[...] + c, 0.0)
    sgn = jnp.sum(z * w2_ref[...], axis=1) + bm2_ref[...]
    o_ref[...] = 1.0 / (1.0 + jnp.exp(-sgn))


_tc4 = pl.pallas_call(
    _tc4_body,
    grid=(pl.cdiv(_E, _RBE),),
    in_specs=[
        pl.BlockSpec((_RBE, _H), lambda i: (i, 0)),
        pl.BlockSpec((_RBE, _DE), lambda i: (i, 0)),
        pl.BlockSpec((_DE, _H), lambda i: (0, 0)),
        pl.BlockSpec((1, _H), lambda i: (0, 0)),
        pl.BlockSpec((1,), lambda i: (0,)),
    ],
    out_specs=pl.BlockSpec((_RBE,), lambda i: (i,)),
    out_shape=jax.ShapeDtypeStruct((_E,), jnp.float32),
)


def kernel(x, edge_index, edge_attr, W1, b1, W2, b2, Wm1, bm1, Wm2, bm2):
    src = edge_index[0]
    dst = edge_index[1]
    src3 = src.reshape(_NW, _NCH, _CH)
    dst3 = dst.reshape(_NW, _NCH, _CH)
    src4 = src.reshape(_NW, _NSS, _SS, _CH)
    dst4 = dst.reshape(_NW, _NSS, _SS, _CH)
    dst2 = dst.reshape(_NW, _EW)
    zeros_nh = jnp.zeros((_N, _H), jnp.float32)

    degp = _deg_call(dst2)
    y1, dinv = _tc1(x, W1, degp.T)
    p1 = _seg_call(y1, zeros_nh, src4, dst4)
    y2 = _tc2(p1, dinv, b1, W2)
    p2 = _seg_call(y2, zeros_nh, src4, dst4)
    a_t, b_t = _tc3(p2, dinv, b2, Wm1[:_H], Wm1[_H:2 * _H], bm1)
    t = _edge_call(a_t, b_t, src3, dst3).reshape(_E, _H)
    return _tc4(t, edge_attr, Wm1[2 * _H:], Wm2.reshape(1, _H), bm2)


# speedup vs baseline: 1.4787x; 1.0528x over previous
import jax
import jax.numpy as jnp
from jax import lax
from jax.experimental import pallas as pl
from jax.experimental.pallas import tpu as pltpu
from jax.experimental.pallas import tpu_sc as plsc

_N = 10000
_E = 320000
_D = 128
_DE = 16
_H = 128

_NC = 2
_NS = 16
_NW = _NC * _NS
_EW = _E // _NW
_CH = 80
_NCH = _EW // _CH
_RPT = 624
_RREM = _N - _NS * _RPT
_SS = 25
_NSS = _NCH // _SS

_mesh = plsc.VectorSubcoreMesh(core_axis_name="c", subcore_axis_name="s")


def _deg_body(dst_hbm, out_hbm, idx_v, acc_v):
    w = lax.axis_index("s") * _NC + lax.axis_index("c")
    zeros = jnp.zeros((16,), jnp.float32)

    def zero(i, c):
        acc_v[pl.ds(i * 16, 16)] = zeros
        return c

    lax.fori_loop(0, _N // 16, zero, 0)
    pltpu.sync_copy(dst_hbm.at[w], idx_v)
    ones = jnp.ones((16,), jnp.float32)

    def body(i, c):
        idx = idx_v[pl.ds(i * 16, 16)]
        plsc.addupdate_scatter(acc_v, [idx], ones)
        return c

    lax.fori_loop(0, _EW // 16, body, 0)
    pltpu.sync_copy(acc_v, out_hbm.at[w])


_deg_call = pl.kernel(
    _deg_body,
    out_type=jax.ShapeDtypeStruct((_NW, _N), jnp.float32),
    mesh=_mesh,
    compiler_params=pltpu.CompilerParams(needs_layout_passes=False),
    scratch_types=[
        pltpu.VMEM((_EW,), jnp.int32),
        pltpu.VMEM((_N,), jnp.float32),
    ],
)


def _seg_body(y_hbm, z_hbm, sidx_hbm, didx_hbm, out_hbm,
              sidx_v, didx_v, b0, b1, b2, b3, acc_sh, sm0, sm1, sm2, sm3):
    c = lax.axis_index("c")
    s = lax.axis_index("s")
    w = s * _NC + c
    r0 = s * _RPT

    @pl.when(c == 0)
    def _():
        pltpu.sync_copy(y_hbm.at[pl.ds(r0, _RPT)], acc_sh.at[pl.ds(r0, _RPT)])

    @pl.when(c != 0)
    def _():
        pltpu.sync_copy(z_hbm.at[pl.ds(r0, _RPT)], acc_sh.at[pl.ds(r0, _RPT)])

    rtail = _NS * _RPT

    @pl.when((c == 0) & (s == 0))
    def _():
        pltpu.sync_copy(y_hbm.at[pl.ds(rtail, _RREM)],
                        acc_sh.at[pl.ds(rtail, _RREM)])

    @pl.when((c != 0) & (s == 0))
    def _():
        pltpu.sync_copy(z_hbm.at[pl.ds(rtail, _RREM)],
                        acc_sh.at[pl.ds(rtail, _RREM)])

    plsc.subcore_barrier()

    bufs = (b0, b1, b2, b3)
    sems = (sm0, sm1, sm2, sm3)

    def gath(j, i):
        pltpu.async_copy(y_hbm.at[sidx_v.at[j]], bufs[i], sems[i])

    def wait(j, i):
        pltpu.make_async_copy(y_hbm.at[sidx_v.at[j]], bufs[i], sems[i]).wait()

    def scat(j, i):
        pltpu.sync_copy(bufs[i], acc_sh.at[didx_v.at[j]], add=True)

    def super_body(ss, carry):
        pltpu.sync_copy(sidx_hbm.at[w, ss], sidx_v)
        pltpu.sync_copy(didx_hbm.at[w, ss], didx_v)
        for o in range(4):
            gath(o, o)

        def body(k, c2):
            for o in range(4):
                j = 4 * k + o
                wait(j, o)
                scat(j, o)
                gath(j + 4, o)
            return c2

        lax.fori_loop(0, _SS // 4 - 1, body, 0)
        wait(_SS - 5, 0)
        scat(_SS - 5, 0)
        gath(_SS - 1, 0)
        wait(_SS - 4, 1)
        scat(_SS - 4, 1)
        wait(_SS - 3, 2)
        scat(_SS - 3, 2)
        wait(_SS - 2, 3)
        scat(_SS - 2, 3)
        wait(_SS - 1, 0)
        scat(_SS - 1, 0)
        return carry

    lax.fori_loop(0, _NSS, super_body, 0)
    plsc.subcore_barrier()
    pltpu.sync_copy(acc_sh.at[pl.ds(r0, _RPT)], out_hbm.at[c, pl.ds(r0, _RPT)])

    @pl.when(s == 0)
    def _():
        pltpu.sync_copy(acc_sh.at[pl.ds(rtail, _RREM)],
                        out_hbm.at[c, pl.ds(rtail, _RREM)])


_seg_call = pl.kernel(
    _seg_body,
    out_type=jax.ShapeDtypeStruct((_NC, _N, _H), jnp.float32),
    mesh=_mesh,
    scratch_types=[
        pltpu.VMEM((_SS, _CH), jnp.int32),
        pltpu.VMEM((_SS, _CH), jnp.int32),
        pltpu.VMEM((_CH, _H), jnp.float32),
        pltpu.VMEM((_CH, _H), jnp.float32),
        pltpu.VMEM((_CH, _H), jnp.float32),
        pltpu.VMEM((_CH, _H), jnp.float32),
        pltpu.VMEM_SHARED((_N, _H), jnp.float32),
        pltpu.SemaphoreType.DMA,
        pltpu.SemaphoreType.DMA,
        pltpu.SemaphoreType.DMA,
        pltpu.SemaphoreType.DMA,
    ],
)


def _edge_body(a_hbm, b_hbm, sidx_hbm, didx_hbm, out_hbm,
               sidx_v, didx_v, b0, b1, b2, b3,
               sa0, sa1, sa2, sa3, sb0, sb1, sb2, sb3):
    w = lax.axis_index("s") * _NC + lax.axis_index("c")
    pltpu.sync_copy(sidx_hbm.at[w], sidx_v)
    pltpu.sync_copy(didx_hbm.at[w], didx_v)
    base = w * _NCH
    bufs = (b0, b1, b2, b3)
    sas = (sa0, sa1, sa2, sa3)
    sbs = (sb0, sb1, sb2, sb3)

    def ga(j, i, sem=None):
        pltpu.async_copy(a_hbm.at[sidx_v.at[j]], bufs[i], sas[i])

    def wa(j, i):
        pltpu.make_async_copy(a_hbm.at[sidx_v.at[j]], bufs[i], sas[i]).wait()

    def gb(j, i):
        pltpu.async_copy(b_hbm.at[didx_v.at[j]], bufs[i], sbs[i], add=True)

    def wb(j, i):
        pltpu.make_async_copy(b_hbm.at[didx_v.at[j]], bufs[i], sbs[i]).wait()

    def wr(j, i):
        pltpu.sync_copy(bufs[i], out_hbm.at[base + j])

    ga(0, 0)
    ga(1, 1)
    wa(0, 0)
    gb(0, 0)
    ga(2, 2)
    wa(1, 1)
    gb(1, 1)
    ga(3, 3)

    def body(k, carry):
        jb = 4 * k + 2
        for o in range(4):
            j = jb + o
            y = (o + 2) % 4
            wb(j - 2, o)
            wr(j - 2, o)
            ga(j + 2, o)
            wa(j, y)
            gb(j, y)
        return carry

    lax.fori_loop(0, (_NCH - 5) // 4, body, 0)
    j = _NCH - 3
    wb(j - 2, 0)
    wr(j - 2, 0)
    ga(j + 2, 0)
    wa(j, 2)
    gb(j, 2)
    j = _NCH - 2
    wb(j - 2, 1)
    wr(j - 2, 1)
    wa(j, 3)
    gb(j, 3)
    j = _NCH - 1
    wb(j - 2, 2)
    wr(j - 2, 2)
    wa(j, 0)
    gb(j, 0)
    wb(_NCH - 2, 3)
    wr(_NCH - 2, 3)
    wb(_NCH - 1, 0)
    wr(_NCH - 1, 0)


_edge_call = pl.kernel(
    _edge_body,
    out_type=jax.ShapeDtypeStruct((_NW * _NCH, _CH, _H), jnp.float32),
    mesh=_mesh,
    scratch_types=[
        pltpu.VMEM((_NCH, _CH), jnp.int32),
        pltpu.VMEM((_NCH, _CH), jnp.int32),
        pltpu.VMEM((_CH, _H), jnp.float32),
        pltpu.VMEM((_CH, _H), jnp.float32),
        pltpu.VMEM((_CH, _H), jnp.float32),
        pltpu.VMEM((_CH, _H), jnp.float32),
        pltpu.SemaphoreType.DMA,
        pltpu.SemaphoreType.DMA,
        pltpu.SemaphoreType.DMA,
        pltpu.SemaphoreType.DMA,
        pltpu.SemaphoreType.DMA,
        pltpu.SemaphoreType.DMA,
        pltpu.SemaphoreType.DMA,
        pltpu.SemaphoreType.DMA,
    ],
)


_RB = 2000
_RBE = 8192


def _tc1_body(x_ref, w_ref, degt_ref, y_ref, dinv_ref):
    deg = jnp.sum(degt_ref[...], axis=1) + 1.0
    dinv = lax.rsqrt(deg)[:, None]
    dinv_ref[...] = dinv
    xw = jnp.dot(x_ref[...], w_ref[...], preferred_element_type=jnp.float32)
    y_ref[...] = xw * dinv


_tc1 = pl.pallas_call(
    _tc1_body,
    grid=(_N // _RB,),
    in_specs=[
        pl.BlockSpec((_RB, _D), lambda i: (i, 0)),
        pl.BlockSpec((_D, _H), lambda i: (0, 0)),
        pl.BlockSpec((_RB, _NW), lambda i: (i, 0)),
    ],
    out_specs=[
        pl.BlockSpec((_RB, _H), lambda i: (i, 0)),
        pl.BlockSpec((_RB, 1), lambda i: (i, 0)),
    ],
    out_shape=[
        jax.ShapeDtypeStruct((_N, _H), jnp.float32),
        jax.ShapeDtypeStruct((_N, 1), jnp.float32),
    ],
)


def _tc2_body(p_ref, dinv_ref, b1_ref, w2_ref, y2_ref):
    dinv = dinv_ref[...]
    h1 = jnp.maximum((p_ref[0] + p_ref[1]) * dinv + b1_ref[...], 0.0)
    y2_ref[...] = jnp.dot(h1, w2_ref[...],
                          preferred_element_type=jnp.float32) * dinv


_tc2 = pl.pallas_call(
    _tc2_body,
    grid=(_N // _RB,),
    in_specs=[
        pl.BlockSpec((_NC, _RB, _H), lambda i: (0, i, 0)),
        pl.BlockSpec((_RB, 1), lambda i: (i, 0)),
        pl.BlockSpec((_H,), lambda i: (0,)),
        pl.BlockSpec((_H, _H), lambda i: (0, 0)),
    ],
    out_specs=pl.BlockSpec((_RB, _H), lambda i: (i, 0)),
    out_shape=jax.ShapeDtypeStruct((_N, _H), jnp.float32),
)


def _tc3_body(q_ref, dinv_ref, b2_ref, wa_ref, wb_ref, bm1_ref, a_ref, b_ref):
    dinv = dinv_ref[...]
    h = (q_ref[0] + q_ref[1]) * dinv + b2_ref[...]
    a_ref[...] = jnp.dot(h, wa_ref[...], preferred_element_type=jnp.float32)
    b_ref[...] = jnp.dot(h, wb_ref[...],
                         preferred_element_type=jnp.float32) + bm1_ref[...]


_tc3 = pl.pallas_call(
    _tc3_body,
    grid=(_N // _RB,),
    in_specs=[
        pl.BlockSpec((_NC, _RB, _H), lambda i: (0, i, 0)),
        pl.BlockSpec((_RB, 1), lambda i: (i, 0)),
        pl.BlockSpec((_H,), lambda i: (0,)),
        pl.BlockSpec((_H, _H), lambda i: (0, 0)),
        pl.BlockSpec((_H, _H), lambda i: (0, 0)),
        pl.BlockSpec((_H,), lambda i: (0,)),
    ],
    out_specs=[
        pl.BlockSpec((_RB, _H), lambda i: (i, 0)),
        pl.BlockSpec((_RB, _H), lambda i: (i, 0)),
    ],
    out_shape=[
        jax.ShapeDtypeStruct((_N, _H), jnp.float32),
        jax.ShapeDtypeStruct((_N, _H), jnp.float32),
    ],
)


def _tc4_body(t_ref, ea_ref, wc_ref, w2_ref, bm2_ref, o_ref):
    c = jnp.dot(ea_ref[...], wc_ref[...], preferred_element_type=jnp.float32)
    z = jnp.maximum(t_ref[...] + c, 0.0)
    sgn = jnp.sum(z * w2_ref[...], axis=1) + bm2_ref[...]
    o_ref[...] = 1.0 / (1.0 + jnp.exp(-sgn))


_tc4 = pl.pallas_call(
    _tc4_body,
    grid=(pl.cdiv(_E, _RBE),),
    in_specs=[
        pl.BlockSpec((_RBE, _H), lambda i: (i, 0)),
        pl.BlockSpec((_RBE, _DE), lambda i: (i, 0)),
        pl.BlockSpec((_DE, _H), lambda i: (0, 0)),
        pl.BlockSpec((1, _H), lambda i: (0, 0)),
        pl.BlockSpec((1,), lambda i: (0,)),
    ],
    out_specs=pl.BlockSpec((_RBE,), lambda i: (i,)),
    out_shape=jax.ShapeDtypeStruct((_E,), jnp.float32),
)


def kernel(x, edge_index, edge_attr, W1, b1, W2, b2, Wm1, bm1, Wm2, bm2):
    src = edge_index[0]
    dst = edge_index[1]
    src3 = src.reshape(_NW, _NCH, _CH)
    dst3 = dst.reshape(_NW, _NCH, _CH)
    src4 = src.reshape(_NW, _NSS, _SS, _CH)
    dst4 = dst.reshape(_NW, _NSS, _SS, _CH)
    dst2 = dst.reshape(_NW, _EW)
    zeros_nh = jnp.zeros((_N, _H), jnp.float32)

    degp = _deg_call(dst2)
    y1, dinv = _tc1(x, W1, degp.T)
    p1 = _seg_call(y1, zeros_nh, src4, dst4)
    y2 = _tc2(p1, dinv, b1, W2)
    p2 = _seg_call(y2, zeros_nh, src4, dst4)
    a_t, b_t = _tc3(p2, dinv, b2, Wm1[:_H], Wm1[_H:2 * _H], bm1)
    t = _edge_call(a_t, b_t, src3, dst3).reshape(_E, _H)
    return _tc4(t, edge_attr, Wm1[2 * _H:], Wm2.reshape(1, _H), bm2)


# speedup vs baseline: 1.5208x; 1.0285x over previous
import jax
import jax.numpy as jnp
from jax import lax
from jax.experimental import pallas as pl
from jax.experimental.pallas import tpu as pltpu
from jax.experimental.pallas import tpu_sc as plsc

_N = 10000
_E = 320000
_D = 128
_DE = 16
_H = 128

_NC = 2
_NS = 16
_NW = _NC * _NS
_EW = _E // _NW
_CH = 80
_NCH = _EW // _CH
_RPT = 624
_RREM = _N - _NS * _RPT
_SS = 25
_NSS = _NCH // _SS

_mesh = plsc.VectorSubcoreMesh(core_axis_name="c", subcore_axis_name="s")


def _deg_body(dst_hbm, out_hbm, idx_v, acc_v):
    w = lax.axis_index("s") * _NC + lax.axis_index("c")
    zeros = jnp.zeros((16,), jnp.float32)

    def zero(i, c):
        acc_v[pl.ds(i * 16, 16)] = zeros
        return c

    lax.fori_loop(0, _N // 16, zero, 0)
    pltpu.sync_copy(dst_hbm.at[w], idx_v)
    ones = jnp.ones((16,), jnp.float32)

    def body(i, c):
        idx = idx_v[pl.ds(i * 16, 16)]
        plsc.addupdate_scatter(acc_v, [idx], ones)
        return c

    lax.fori_loop(0, _EW // 16, body, 0)
    pltpu.sync_copy(acc_v, out_hbm.at[w])


_deg_call = pl.kernel(
    _deg_body,
    out_type=jax.ShapeDtypeStruct((_NW, _N), jnp.float32),
    mesh=_mesh,
    compiler_params=pltpu.CompilerParams(needs_layout_passes=False),
    scratch_types=[
        pltpu.VMEM((_EW,), jnp.int32),
        pltpu.VMEM((_N,), jnp.float32),
    ],
)


def _seg_body(y_hbm, z_hbm, sidx_hbm, didx_hbm, out_hbm,
              sidx_v, didx_v, b0, b1, b2, b3, acc_sh, sm0, sm1, sm2, sm3):
    c = lax.axis_index("c")
    s = lax.axis_index("s")
    w = s * _NC + c
    r0 = s * _RPT

    @pl.when(c == 0)
    def _():
        pltpu.sync_copy(y_hbm.at[pl.ds(r0, _RPT)], acc_sh.at[pl.ds(r0, _RPT)])

    @pl.when(c != 0)
    def _():
        pltpu.sync_copy(z_hbm.at[pl.ds(r0, _RPT)], acc_sh.at[pl.ds(r0, _RPT)])

    rtail = _NS * _RPT

    @pl.when((c == 0) & (s == 0))
    def _():
        pltpu.sync_copy(y_hbm.at[pl.ds(rtail, _RREM)],
                        acc_sh.at[pl.ds(rtail, _RREM)])

    @pl.when((c != 0) & (s == 0))
    def _():
        pltpu.sync_copy(z_hbm.at[pl.ds(rtail, _RREM)],
                        acc_sh.at[pl.ds(rtail, _RREM)])

    plsc.subcore_barrier()

    bufs = (b0, b1, b2, b3)
    sems = (sm0, sm1, sm2, sm3)

    def gath(j, i):
        pltpu.async_copy(y_hbm.at[sidx_v.at[j]], bufs[i], sems[i])

    def wait(j, i):
        pltpu.make_async_copy(y_hbm.at[sidx_v.at[j]], bufs[i], sems[i]).wait()

    def scat(j, i):
        pltpu.sync_copy(bufs[i], acc_sh.at[didx_v.at[j]], add=True)

    def super_body(ss, carry):
        pltpu.sync_copy(sidx_hbm.at[w, ss], sidx_v)
        pltpu.sync_copy(didx_hbm.at[w, ss], didx_v)
        for o in range(4):
            gath(o, o)

        def body(k, c2):
            for o in range(4):
                j = 4 * k + o
                wait(j, o)
                scat(j, o)
                gath(j + 4, o)
            return c2

        lax.fori_loop(0, _SS // 4 - 1, body, 0)
        wait(_SS - 5, 0)
        scat(_SS - 5, 0)
        gath(_SS - 1, 0)
        wait(_SS - 4, 1)
        scat(_SS - 4, 1)
        wait(_SS - 3, 2)
        scat(_SS - 3, 2)
        wait(_SS - 2, 3)
        scat(_SS - 2, 3)
        wait(_SS - 1, 0)
        scat(_SS - 1, 0)
        return carry

    lax.fori_loop(0, _NSS, super_body, 0)
    plsc.subcore_barrier()
    pltpu.sync_copy(acc_sh.at[pl.ds(r0, _RPT)], out_hbm.at[c, pl.ds(r0, _RPT)])

    @pl.when(s == 0)
    def _():
        pltpu.sync_copy(acc_sh.at[pl.ds(rtail, _RREM)],
                        out_hbm.at[c, pl.ds(rtail, _RREM)])


_seg_call = pl.kernel(
    _seg_body,
    out_type=jax.ShapeDtypeStruct((_NC, _N, _H), jnp.float32),
    mesh=_mesh,
    scratch_types=[
        pltpu.VMEM((_SS, _CH), jnp.int32),
        pltpu.VMEM((_SS, _CH), jnp.int32),
        pltpu.VMEM((_CH, _H), jnp.float32),
        pltpu.VMEM((_CH, _H), jnp.float32),
        pltpu.VMEM((_CH, _H), jnp.float32),
        pltpu.VMEM((_CH, _H), jnp.float32),
        pltpu.VMEM_SHARED((_N, _H), jnp.float32),
        pltpu.SemaphoreType.DMA,
        pltpu.SemaphoreType.DMA,
        pltpu.SemaphoreType.DMA,
        pltpu.SemaphoreType.DMA,
    ],
)


def _edge_body(a_hbm, b_hbm, sidx_hbm, didx_hbm, out_hbm,
               sidx_v, didx_v, b0, b1, b2, b3,
               sa0, sa1, sa2, sa3, sb0, sb1, sb2, sb3):
    w = lax.axis_index("s") * _NC + lax.axis_index("c")
    pltpu.sync_copy(sidx_hbm.at[w], sidx_v)
    pltpu.sync_copy(didx_hbm.at[w], didx_v)
    base = w * _NCH
    bufs = (b0, b1, b2, b3)
    sas = (sa0, sa1, sa2, sa3)
    sbs = (sb0, sb1, sb2, sb3)

    def ga(j, i, sem=None):
        pltpu.async_copy(a_hbm.at[sidx_v.at[j]], bufs[i], sas[i])

    def wa(j, i):
        pltpu.make_async_copy(a_hbm.at[sidx_v.at[j]], bufs[i], sas[i]).wait()

    def gb(j, i):
        pltpu.async_copy(b_hbm.at[didx_v.at[j]], bufs[i], sbs[i], add=True)

    def wb(j, i):
        pltpu.make_async_copy(b_hbm.at[didx_v.at[j]], bufs[i], sbs[i]).wait()

    def wr(j, i):
        pltpu.sync_copy(bufs[i], out_hbm.at[base + j])

    ga(0, 0)
    ga(1, 1)
    wa(0, 0)
    gb(0, 0)
    ga(2, 2)
    wa(1, 1)
    gb(1, 1)
    ga(3, 3)

    def body(k, carry):
        jb = 4 * k + 2
        for o in range(4):
            j = jb + o
            y = (o + 2) % 4
            wb(j - 2, o)
            wr(j - 2, o)
            ga(j + 2, o)
            wa(j, y)
            gb(j, y)
        return carry

    lax.fori_loop(0, (_NCH - 5) // 4, body, 0)
    j = _NCH - 3
    wb(j - 2, 0)
    wr(j - 2, 0)
    ga(j + 2, 0)
    wa(j, 2)
    gb(j, 2)
    j = _NCH - 2
    wb(j - 2, 1)
    wr(j - 2, 1)
    wa(j, 3)
    gb(j, 3)
    j = _NCH - 1
    wb(j - 2, 2)
    wr(j - 2, 2)
    wa(j, 0)
    gb(j, 0)
    wb(_NCH - 2, 3)
    wr(_NCH - 2, 3)
    wb(_NCH - 1, 0)
    wr(_NCH - 1, 0)


_edge_call = pl.kernel(
    _edge_body,
    out_type=jax.ShapeDtypeStruct((_NW * _NCH, _CH, _H), jnp.float32),
    mesh=_mesh,
    scratch_types=[
        pltpu.VMEM((_NCH, _CH), jnp.int32),
        pltpu.VMEM((_NCH, _CH), jnp.int32),
        pltpu.VMEM((_CH, _H), jnp.float32),
        pltpu.VMEM((_CH, _H), jnp.float32),
        pltpu.VMEM((_CH, _H), jnp.float32),
        pltpu.VMEM((_CH, _H), jnp.float32),
        pltpu.SemaphoreType.DMA,
        pltpu.SemaphoreType.DMA,
        pltpu.SemaphoreType.DMA,
        pltpu.SemaphoreType.DMA,
        pltpu.SemaphoreType.DMA,
        pltpu.SemaphoreType.DMA,
        pltpu.SemaphoreType.DMA,
        pltpu.SemaphoreType.DMA,
    ],
)


_RB = 2000
_RBE = 8192


def _tc1_body(x_ref, w_ref, degt_ref, y_ref, dinv_ref):
    deg = jnp.sum(degt_ref[...], axis=1) + 1.0
    dinv = lax.rsqrt(deg)[:, None]
    dinv_ref[...] = dinv
    xw = jnp.dot(x_ref[...], w_ref[...], preferred_element_type=jnp.float32)
    y_ref[...] = xw * dinv


_tc1 = pl.pallas_call(
    _tc1_body,
    grid=(_N // _RB,),
    in_specs=[
        pl.BlockSpec((_RB, _D), lambda i: (i, 0)),
        pl.BlockSpec((_D, _H), lambda i: (0, 0)),
        pl.BlockSpec((_RB, _NW), lambda i: (i, 0)),
    ],
    out_specs=[
        pl.BlockSpec((_RB, _H), lambda i: (i, 0)),
        pl.BlockSpec((_RB, 1), lambda i: (i, 0)),
    ],
    out_shape=[
        jax.ShapeDtypeStruct((_N, _H), jnp.float32),
        jax.ShapeDtypeStruct((_N, 1), jnp.float32),
    ],
)


def _tc2_body(p_ref, dinv_ref, b1_ref, w2_ref, y2_ref):
    dinv = dinv_ref[...]
    h1 = jnp.maximum((p_ref[0] + p_ref[1]) * dinv + b1_ref[...], 0.0)
    y2_ref[...] = jnp.dot(h1, w2_ref[...],
                          preferred_element_type=jnp.float32) * dinv


_tc2 = pl.pallas_call(
    _tc2_body,
    grid=(_N // _RB,),
    in_specs=[
        pl.BlockSpec((_NC, _RB, _H), lambda i: (0, i, 0)),
        pl.BlockSpec((_RB, 1), lambda i: (i, 0)),
        pl.BlockSpec((_H,), lambda i: (0,)),
        pl.BlockSpec((_H, _H), lambda i: (0, 0)),
    ],
    out_specs=pl.BlockSpec((_RB, _H), lambda i: (i, 0)),
    out_shape=jax.ShapeDtypeStruct((_N, _H), jnp.float32),
)


def _tc3_body(q_ref, dinv_ref, b2_ref, wa_ref, wb_ref, bm1_ref, a_ref, b_ref):
    dinv = dinv_ref[...]
    h = (q_ref[0] + q_ref[1]) * dinv + b2_ref[...]
    a_ref[...] = jnp.dot(h, wa_ref[...], preferred_element_type=jnp.float32)
    b_ref[...] = jnp.dot(h, wb_ref[...],
                         preferred_element_type=jnp.float32) + bm1_ref[...]


_tc3 = pl.pallas_call(
    _tc3_body,
    grid=(_N // _RB,),
    in_specs=[
        pl.BlockSpec((_NC, _RB, _H), lambda i: (0, i, 0)),
        pl.BlockSpec((_RB, 1), lambda i: (i, 0)),
        pl.BlockSpec((_H,), lambda i: (0,)),
        pl.BlockSpec((_H, _H), lambda i: (0, 0)),
        pl.BlockSpec((_H, _H), lambda i: (0, 0)),
        pl.BlockSpec((_H,), lambda i: (0,)),
    ],
    out_specs=[
        pl.BlockSpec((_RB, _H), lambda i: (i, 0)),
        pl.BlockSpec((_RB, _H), lambda i: (i, 0)),
    ],
    out_shape=[
        jax.ShapeDtypeStruct((_N, _H), jnp.float32),
        jax.ShapeDtypeStruct((_N, _H), jnp.float32),
    ],
)


def _tc4_body(t_ref, ea_ref, wc_ref, w2_ref, bm2_ref, o_ref):
    ea = ea_ref[...].astype(jnp.float32)
    c = jnp.dot(ea, wc_ref[...], preferred_element_type=jnp.float32)
    z = jnp.maximum(t_ref[...] + c, 0.0)
    sgn = jnp.sum(z * w2_ref[...], axis=1) + bm2_ref[...]
    o_ref[...] = 1.0 / (1.0 + jnp.exp(-sgn))


_tc4 = pl.pallas_call(
    _tc4_body,
    grid=(pl.cdiv(_E, _RBE),),
    in_specs=[
        pl.BlockSpec((_RBE, _H), lambda i: (i, 0)),
        pl.BlockSpec((_RBE, _DE), lambda i: (i, 0)),
        pl.BlockSpec((_DE, _H), lambda i: (0, 0)),
        pl.BlockSpec((1, _H), lambda i: (0, 0)),
        pl.BlockSpec((1,), lambda i: (0,)),
    ],
    out_specs=pl.BlockSpec((_RBE,), lambda i: (i,)),
    out_shape=jax.ShapeDtypeStruct((_E,), jnp.float32),
)


def kernel(x, edge_index, edge_attr, W1, b1, W2, b2, Wm1, bm1, Wm2, bm2):
    src = edge_index[0]
    dst = edge_index[1]
    src3 = src.reshape(_NW, _NCH, _CH)
    dst3 = dst.reshape(_NW, _NCH, _CH)
    src4 = src.reshape(_NW, _NSS, _SS, _CH)
    dst4 = dst.reshape(_NW, _NSS, _SS, _CH)
    dst2 = dst.reshape(_NW, _EW)
    zeros_nh = jnp.zeros((_N, _H), jnp.float32)

    degp = _deg_call(dst2)
    y1, dinv = _tc1(x, W1, degp.T)
    p1 = _seg_call(y1, zeros_nh, src4, dst4)
    y2 = _tc2(p1, dinv, b1, W2)
    p2 = _seg_call(y2, zeros_nh, src4, dst4)
    a_t, b_t = _tc3(p2, dinv, b2, Wm1[:_H], Wm1[_H:2 * _H], bm1)
    t = _edge_call(a_t, b_t, src3, dst3).reshape(_E, _H)
    return _tc4(t, edge_attr.astype(jnp.bfloat16),
                Wm1[2 * _H:], Wm2.reshape(1, _H), bm2)
